# baseline (speedup 1.0000x reference)
import jax
import jax.numpy as jnp
from jax.experimental import pallas as pl
from jax.experimental.pallas import tpu as pltpu
from jax.experimental.pallas import tpu_sc as plsc

_B = 425984
_D = 32
_W = 128

_vector_mesh = plsc.VectorSubcoreMesh(core_axis_name="core",
                                      subcore_axis_name="subcore")


def _gather_fn(table, indices):
    @pl.kernel(
        out_type=jax.ShapeDtypeStruct((_B, _D), jnp.float32),
        mesh=_vector_mesh,
        compiler_params=pltpu.CompilerParams(use_tc_tiling_on_sc=False),
    )
    def body(x_hbm, i_hbm, o_hbm):
        def step(i_vmem, o_vmem):
            pltpu.sync_copy(x_hbm.at[i_vmem.at[0]], o_vmem)

        pltpu.emit_pipeline(
            step,
            grid=(_B // _W,),
            in_specs=[pl.BlockSpec((1, _W), index_map=lambda i: (0, i))],
            out_specs=[pl.BlockSpec((_W, _D), index_map=lambda i: (i, 0))],
            core_axis_name="subcore",
            dimension_semantics=(pltpu.PARALLEL,),
        )(i_hbm, o_hbm)

    return body(table, indices)


def kernel(variable_embeddings, candidate_indices):
    idx = candidate_indices.astype(jnp.int32).reshape((1, _B))
    return _gather_fn(variable_embeddings, idx)


# speedup vs baseline: 1.1205x; 1.1205x over previous
import jax
import jax.numpy as jnp
from jax.experimental import pallas as pl
from jax.experimental.pallas import tpu as pltpu
from jax.experimental.pallas import tpu_sc as plsc

_B = 425984
_D = 32
_W = 512

_vector_mesh = plsc.VectorSubcoreMesh(core_axis_name="core",
                                      subcore_axis_name="subcore")


def _gather_fn(table, indices):
    @pl.kernel(
        out_type=jax.ShapeDtypeStruct((_B, _D), jnp.float32),
        mesh=_vector_mesh,
        compiler_params=pltpu.CompilerParams(use_tc_tiling_on_sc=False),
    )
    def body(x_hbm, i_hbm, o_hbm):
        def step(i_vmem, o_vmem):
            pltpu.sync_copy(x_hbm.at[i_vmem.at[0]], o_vmem)

        pltpu.emit_pipeline(
            step,
            grid=(_B // _W,),
            in_specs=[pl.BlockSpec((1, _W), index_map=lambda i: (0, i))],
            out_specs=[pl.BlockSpec((_W, _D), index_map=lambda i: (i, 0))],
            core_axis_name="subcore",
            dimension_semantics=(pltpu.PARALLEL,),
        )(i_hbm, o_hbm)

    return body(table, indices)


def kernel(variable_embeddings, candidate_indices):
    idx = candidate_indices.astype(jnp.int32).reshape((1, _B))
    return _gather_fn(variable_embeddings, idx)
